# baseline (speedup 1.0000x reference)
import jax
import jax.numpy as jnp
from jax.experimental import pallas as pl


def _tc_pool(x4, k, c, rows_blk):
    b, n_out, kc = x4.shape

    def body(in_ref, out_ref):
        acc = in_ref[:, :, 0:c]
        for kk in range(1, k):
            acc = acc + in_ref[:, :, kk * c:(kk + 1) * c]
        out_ref[...] = acc * (1.0 / k)

    return pl.pallas_call(
        body,
        grid=(b, n_out // rows_blk),
        in_specs=[pl.BlockSpec((1, rows_blk, kc), lambda i, j: (i, j, 0))],
        out_specs=pl.BlockSpec((1, rows_blk, c), lambda i, j: (i, j, 0)),
        out_shape=jax.ShapeDtypeStruct((b, n_out, c), jnp.float32),
    )(x4)


def kernel(x, connection_indices):
    b, n_in, c = x.shape
    n_out, k = connection_indices.shape
    x4 = x.reshape(b, n_out, k * c)
    return _tc_pool(x4, int(k), int(c), 1024)
